# baseline (speedup 1.0000x reference)
import jax
import jax.numpy as jnp
from jax.experimental import pallas as pl

N = 10000
GAMMA = -0.1
ZETA = 1.1


def _spmm(idx, vals, h):
    return jax.ops.segment_sum(vals[:, None] * h[idx[1]], idx[0], num_segments=N)


def kernel(x, adj_indices, adj_values, gcn_w, gcn_b, mean_w1, mean_b1, mean_w2, mean_b2, std_w1, std_b1, std_w2, std_b2, den_nb_w, den_nb_b, den_self_w, den_self_b, den_att_w, den_att_b, fus_a, fus_b, fus_alpha, head_w, head_b):
    def encoder(vals, noise_key):
        h = jax.nn.relu(_spmm(adj_indices, vals, x @ gcn_w + gcn_b))
        mean = jax.nn.relu(h @ mean_w1 + mean_b1) @ mean_w2 + mean_b2
        std = jax.nn.softplus(jax.nn.relu(h @ std_w1 + std_b1) @ std_w2 + std_b2)
        noise = jax.random.normal(noise_key, mean.shape, mean.dtype)
        z = noise * std + mean
        return z, mean, std

    z_gen, mu, x_std = encoder(adj_values, jax.random.key(11))
    adj_logits = z_gen @ z_gen.T

    row, col = adj_indices[0], adj_indices[1]
    s1 = (jax.nn.relu(x @ den_nb_w + den_nb_b) @ den_att_w[:256])[:, 0]
    s2 = (jax.nn.relu(x @ den_self_w + den_self_b) @ den_att_w[256:])[:, 0]
    weight = s1[row] + s2[col] + den_att_b[0]
    rn = jax.random.uniform(jax.random.key(7), (weight.shape[0], 1), weight.dtype) + 1e-07
    logit_rn = (jnp.log(rn) - jnp.log(1.0 - rn))[:, 0]
    gate = jax.nn.sigmoid(logit_rn + weight)
    stretched = gate * (ZETA - GAMMA) + GAMMA
    mask = jnp.clip(stretched, 0.0, 1.0)
    new_vals = adj_values * mask
    l0_loss = jnp.mean(jax.nn.sigmoid(weight - jnp.log(jnp.asarray(-GAMMA / ZETA, weight.dtype))))

    z_den, _, _ = encoder(new_vals, jax.random.key(13))

    z_i = fus_a * z_gen + fus_b * z_den
    z_l = _spmm(adj_indices, adj_values, z_i)
    z_fused = fus_alpha * z_l + (1.0 - fus_alpha) * z_i

    q_fused = jax.nn.softmax(z_fused @ head_w + head_b, axis=1)
    q_gen = jax.nn.softmax(z_gen @ head_w + head_b, axis=1)
    q_den = jax.nn.softmax(z_den @ head_w + head_b, axis=1)

    return (q_fused, q_gen, q_den, adj_logits, z_fused, mu, x_std, l0_loss)


# speedup vs baseline: 2.9413x; 2.9413x over previous
import functools
import math

import jax
import jax.numpy as jnp
from jax import lax
from jax.experimental import pallas as pl
from jax.experimental.pallas import tpu as pltpu
from jax.experimental.pallas import tpu_sc as plsc

N = 10000
D = 256
H = 256
Z = 256
C = 16
E = 160000
GAMMA = -0.1
ZETA = 1.1

RB = 1000
NTILES = 16
EPT = E // NTILES
CH = 80
SB = 5
NSC = EPT // (SB * CH)
NW = E // (SB * CH)
NP = 10240
RPT = NP // NTILES
Q = 64


def _ka_body(x_ref, wcat_ref, bcat_ref, a1_ref, a2_ref,
             xw4_ref, s1_ref, s2_ref):
    u = jnp.dot(x_ref[...], wcat_ref[...], preferred_element_type=jnp.float32)
    u = u + bcat_ref[...][None, :]
    xw = u[:, :256]
    t1 = jnp.maximum(u[:, 256:512], 0.0)
    t2 = jnp.maximum(u[:, 512:768], 0.0)
    for q in range(4):
        xw4_ref[q] = xw[:, q * 64:(q + 1) * 64]
    s1_ref[...] = jnp.sum(t1 * a1_ref[...], axis=1)[:, None]
    s2_ref[...] = jnp.sum(t2 * a2_ref[...], axis=1)[:, None]


def _run_ka(x, wcat, bcat, a1, a2):
    grid = N // RB
    qspec = pl.BlockSpec((4, RB, Q), lambda i: (0, i, 0))
    return pl.pallas_call(
        _ka_body,
        grid=(grid,),
        in_specs=[
            pl.BlockSpec((RB, 256), lambda i: (i, 0)),
            pl.BlockSpec((256, 768), lambda i: (0, 0)),
            pl.BlockSpec((768,), lambda i: (0,)),
            pl.BlockSpec((1, 256), lambda i: (0, 0)),
            pl.BlockSpec((1, 256), lambda i: (0, 0)),
        ],
        out_specs=[qspec,
                   pl.BlockSpec((RB, 1), lambda i: (i, 0)),
                   pl.BlockSpec((RB, 1), lambda i: (i, 0))],
        out_shape=[jax.ShapeDtypeStruct((4, N, Q), jnp.float32)]
        + [jax.ShapeDtypeStruct((N, 1), jnp.float32)] * 2,
    )(x, wcat, bcat, a1, a2)


L0C = math.log(0.1 / 1.1)


def _sigmoid16(t):
    return 1.0 / (1.0 + jnp.exp(-t))


def _sc1_body(xwf, row3d, col3d, val3d, lr3d, s1_hbm, s2_hbm,
              hg4, hd4, l0p, nv3d,
              s1v, s2v, rowc, colc, valc, lrc, nvc, colq,
              rowbuf, sgb, sdb, l0v, acc_g, acc_d, sem):
    c = lax.axis_index("c")
    s = lax.axis_index("s")

    pltpu.sync_copy(s1_hbm, s1v)
    pltpu.sync_copy(s2_hbm, s2v)
    l0v[...] = jnp.zeros((16,), jnp.float32)

    for p in range(2):
        qdyn = 2 * c + p

        def zrow(i, _):
            for u in range(Q // 16):
                sgb[i, pl.ds(u * 16, 16)] = jnp.zeros((16,), jnp.float32)
            return 0

        lax.fori_loop(0, CH, zrow, 0)
        for i in range(RPT // CH):
            pltpu.sync_copy(sgb, acc_g.at[pl.ds(s * RPT + i * CH, CH)])
            pltpu.sync_copy(sgb, acc_d.at[pl.ds(s * RPT + i * CH, CH)])
        plsc.subcore_barrier()

        def superchunk(wl, _):
            w = s * NSC + wl
            pltpu.sync_copy(row3d.at[w], rowc)
            pltpu.sync_copy(col3d.at[w], colc)
            pltpu.sync_copy(val3d.at[w], valc)
            if p == 0:
                pltpu.sync_copy(lr3d.at[w], lrc)

                def gate_chunk(m, acc):
                    def gate_group(g, a):
                        r16 = rowc[m, pl.ds(g * 16, 16)]
                        c16 = colc[m, pl.ds(g * 16, 16)]
                        wgt = (plsc.load_gather(s1v, [r16])
                               + plsc.load_gather(s2v, [c16]))
                        gate = _sigmoid16(lrc[m, pl.ds(g * 16, 16)] + wgt)
                        msk = jnp.clip(gate * (ZETA - GAMMA) + GAMMA, 0.0, 1.0)
                        nvc[m, pl.ds(g * 16, 16)] = (
                            valc[m, pl.ds(g * 16, 16)] * msk)
                        return a + _sigmoid16(wgt - L0C)

                    return lax.fori_loop(0, CH // 16, gate_group, acc)

                acc = lax.fori_loop(0, SB, gate_chunk,
                                    jnp.zeros((16,), jnp.float32))
                l0v[...] = l0v[...] + acc
                pltpu.sync_copy(nvc, nv3d.at[c * NW + w])
            else:
                pltpu.sync_copy(nv3d.at[c * NW + w], nvc)

            for m in range(SB):
                def addq(g, _):
                    colq[pl.ds(g * 16, 16)] = (
                        colc[m, pl.ds(g * 16, 16)] + qdyn * N)
                    return 0

                lax.fori_loop(0, CH // 16, addq, 0)
                pltpu.async_copy(xwf.at[colq], rowbuf, sem).wait()

                def scale(k, _):
                    vg = plsc.load_gather(valc, [jnp.full((16,), m, jnp.int32),
                                                 jnp.full((16,), k, jnp.int32)])
                    vd = plsc.load_gather(nvc, [jnp.full((16,), m, jnp.int32),
                                                jnp.full((16,), k, jnp.int32)])
                    for u in range(Q // 16):
                        r = rowbuf[k, pl.ds(u * 16, 16)]
                        sgb[k, pl.ds(u * 16, 16)] = r * vg
                        sdb[k, pl.ds(u * 16, 16)] = r * vd
                    return 0

                lax.fori_loop(0, CH, scale, 0)
                pltpu.sync_copy(sgb, acc_g.at[rowc.at[m]], add=True)
                pltpu.sync_copy(sdb, acc_d.at[rowc.at[m]], add=True)
            return 0

        lax.fori_loop(0, NSC, superchunk, 0)
        plsc.subcore_barrier()

        for i in range(RPT // CH):
            r0 = s * RPT + i * CH
            pltpu.sync_copy(acc_g.at[pl.ds(r0, CH)], sgb)
            pltpu.sync_copy(sgb, hg4.at[qdyn, pl.ds(r0, CH)])
            pltpu.sync_copy(acc_d.at[pl.ds(r0, CH)], sdb)
            pltpu.sync_copy(sdb, hd4.at[qdyn, pl.ds(r0, CH)])
        if p == 0:
            plsc.subcore_barrier()

    pltpu.sync_copy(l0v, l0p.at[c * 16 + s, 0])


def _run_sc1(xwf, row3d, col3d, val3d, lr3d, s1, s2):
    mesh = plsc.VectorSubcoreMesh(core_axis_name="c", subcore_axis_name="s")
    f = pl.kernel(
        _sc1_body,
        compiler_params=pltpu.CompilerParams(
            use_tc_tiling_on_sc=False, needs_layout_passes=False),
        out_type=[
            jax.ShapeDtypeStruct((4, NP, Q), jnp.float32),
            jax.ShapeDtypeStruct((4, NP, Q), jnp.float32),
            jax.ShapeDtypeStruct((32, 1, 16), jnp.float32),
            jax.ShapeDtypeStruct((2 * NW, SB, CH), jnp.float32),
        ],
        mesh=mesh,
        scratch_types=[
            pltpu.VMEM((N,), jnp.float32),
            pltpu.VMEM((N,), jnp.float32),
            pltpu.VMEM((SB, CH), jnp.int32),
            pltpu.VMEM((SB, CH), jnp.int32),
            pltpu.VMEM((SB, CH), jnp.float32),
            pltpu.VMEM((SB, CH), jnp.float32),
            pltpu.VMEM((SB, CH), jnp.float32),
            pltpu.VMEM((CH,), jnp.int32),
            pltpu.VMEM((CH, Q), jnp.float32),
            pltpu.VMEM((CH, Q), jnp.float32),
            pltpu.VMEM((CH, Q), jnp.float32),
            pltpu.VMEM((16,), jnp.float32),
            pltpu.VMEM_SHARED((NP, Q), jnp.float32),
            pltpu.VMEM_SHARED((NP, Q), jnp.float32),
            pltpu.SemaphoreType.DMA,
        ],
    )
    return f(xwf, row3d, col3d, val3d, lr3d, s1, s2)


def _sc2_body(zif, row3d, col3d, val3d,
              zl2,
              rowc, colc, valc, colq, rowbuf, sgb, acc, sem):
    c = lax.axis_index("c")
    s = lax.axis_index("s")
    QH = 2 * Q

    def zrow(i, _):
        for u in range(QH // 16):
            sgb[i, pl.ds(u * 16, 16)] = jnp.zeros((16,), jnp.float32)
        return 0

    lax.fori_loop(0, CH, zrow, 0)
    for i in range(RPT // CH):
        pltpu.sync_copy(sgb, acc.at[pl.ds(s * RPT + i * CH, CH)])
    plsc.subcore_barrier()

    def superchunk(wl, _):
        w = s * NSC + wl
        pltpu.sync_copy(row3d.at[w], rowc)
        pltpu.sync_copy(col3d.at[w], colc)
        pltpu.sync_copy(val3d.at[w], valc)
        for m in range(SB):
            def addq(g, _):
                colq[pl.ds(g * 16, 16)] = (
                    colc[m, pl.ds(g * 16, 16)] + c * N)
                return 0

            lax.fori_loop(0, CH // 16, addq, 0)
            pltpu.async_copy(zif.at[colq], rowbuf, sem).wait()

            def scale(k, _):
                vg = plsc.load_gather(valc, [jnp.full((16,), m, jnp.int32),
                                             jnp.full((16,), k, jnp.int32)])
                for u in range(QH // 16):
                    sgb[k, pl.ds(u * 16, 16)] = rowbuf[k, pl.ds(u * 16, 16)] * vg
                return 0

            lax.fori_loop(0, CH, scale, 0)
            pltpu.sync_copy(sgb, acc.at[rowc.at[m]], add=True)
        return 0

    lax.fori_loop(0, NSC, superchunk, 0)
    plsc.subcore_barrier()

    for i in range(RPT // CH):
        r0 = s * RPT + i * CH
        pltpu.sync_copy(acc.at[pl.ds(r0, CH)], sgb)
        pltpu.sync_copy(sgb, zl2.at[c, pl.ds(r0, CH)])


def _run_sc2(zif, row3d, col3d, val3d):
    mesh = plsc.VectorSubcoreMesh(core_axis_name="c", subcore_axis_name="s")
    f = pl.kernel(
        _sc2_body,
        compiler_params=pltpu.CompilerParams(
            use_tc_tiling_on_sc=False, needs_layout_passes=False),
        out_type=[jax.ShapeDtypeStruct((2, NP, 2 * Q), jnp.float32)],
        mesh=mesh,
        scratch_types=[
            pltpu.VMEM((SB, CH), jnp.int32),
            pltpu.VMEM((SB, CH), jnp.int32),
            pltpu.VMEM((SB, CH), jnp.float32),
            pltpu.VMEM((CH,), jnp.int32),
            pltpu.VMEM((CH, 2 * Q), jnp.float32),
            pltpu.VMEM((CH, 2 * Q), jnp.float32),
            pltpu.VMEM_SHARED((NP, 2 * Q), jnp.float32),
            pltpu.SemaphoreType.DMA,
        ],
    )
    return f(zif, row3d, col3d, val3d)


def _softplus(t):
    return jnp.maximum(t, 0.0) + jnp.log1p(jnp.exp(-jnp.abs(t)))


def _kb_body(hg_ref, hd_ref, mw1_ref, mb1_ref, mw2_ref, mb2_ref,
             sw1_ref, sb1_ref, sw2_ref, sb2_ref, ng_ref, nd_ref,
             fa_ref, fb_ref,
             zg_ref, zd_ref, mu_ref, std_ref,
             zi0_ref, zi1_ref):
    def enc(h):
        m1 = jnp.maximum(
            jnp.dot(h, mw1_ref[...], preferred_element_type=jnp.float32)
            + mb1_ref[...][None, :], 0.0)
        mean = (jnp.dot(m1, mw2_ref[...], preferred_element_type=jnp.float32)
                + mb2_ref[...][None, :])
        s1 = jnp.maximum(
            jnp.dot(h, sw1_ref[...], preferred_element_type=jnp.float32)
            + sb1_ref[...][None, :], 0.0)
        std = _softplus(
            jnp.dot(s1, sw2_ref[...], preferred_element_type=jnp.float32)
            + sb2_ref[...][None, :])
        return mean, std

    hg = jnp.concatenate(
        [jnp.maximum(hg_ref[q], 0.0) for q in range(4)], axis=1)
    hd = jnp.concatenate(
        [jnp.maximum(hd_ref[q], 0.0) for q in range(4)], axis=1)
    mu, std_g = enc(hg)
    mu_d, std_d = enc(hd)
    zg = ng_ref[...] * std_g + mu
    zd = nd_ref[...] * std_d + mu_d
    zi = fa_ref[...] * zg + fb_ref[...] * zd
    zg_ref[...] = zg
    zd_ref[...] = zd
    mu_ref[...] = mu
    std_ref[...] = std_g
    zi0_ref[...] = zi[:, 0:128]
    zi1_ref[...] = zi[:, 128:256]


def _run_kb(hg4, hd4, mw1, mb1, mw2, mb2, sw1, sb1, sw2, sb2, ng, nd, fa, fb):
    grid = N // RB
    h4spec = pl.BlockSpec((4, RB, Q), lambda i: (0, i, 0))
    wspec = pl.BlockSpec((256, 256), lambda i: (0, 0))
    bspec = pl.BlockSpec((256,), lambda i: (0,))
    nspec = pl.BlockSpec((RB, 256), lambda i: (i, 0))
    hspec = pl.BlockSpec((RB, 128), lambda i: (i, 0))
    return pl.pallas_call(
        _kb_body,
        grid=(grid,),
        in_specs=[h4spec, h4spec, wspec, bspec, wspec, bspec,
                  wspec, bspec, wspec, bspec, nspec, nspec, nspec, nspec],
        out_specs=[nspec, nspec, nspec, nspec, hspec, hspec],
        out_shape=[jax.ShapeDtypeStruct((N, 256), jnp.float32)] * 4
        + [jax.ShapeDtypeStruct((N, 128), jnp.float32)] * 2,
    )(hg4, hd4, mw1, mb1, mw2, mb2, sw1, sb1, sw2, sb2, ng, nd, fa, fb)


def _softmax(t):
    m = jnp.max(t, axis=1, keepdims=True)
    e = jnp.exp(t - m)
    return e / jnp.sum(e, axis=1, keepdims=True)


def _kd_body(zl2_ref, zg_ref, zd_ref, zi_ref, alpha_ref, hw_ref, hb_ref,
             zf_ref, qf_ref, qg_ref, qd_ref):
    zl = jnp.concatenate([zl2_ref[q] for q in range(2)], axis=1)
    alpha = alpha_ref[0]
    zi = zi_ref[...]
    zf = alpha * zl + (1.0 - alpha) * zi
    zf_ref[...] = zf
    hw = hw_ref[...]
    hb = hb_ref[...][None, :]
    qf_ref[...] = _softmax(
        jnp.dot(zf, hw, preferred_element_type=jnp.float32) + hb)
    qg_ref[...] = _softmax(
        jnp.dot(zg_ref[...], hw, preferred_element_type=jnp.float32) + hb)
    qd_ref[...] = _softmax(
        jnp.dot(zd_ref[...], hw, preferred_element_type=jnp.float32) + hb)


def _run_kd(zl2, zg, zd, zi, alpha, hw, hb):
    grid = N // RB
    nspec = pl.BlockSpec((RB, 256), lambda i: (i, 0))
    cspec = pl.BlockSpec((RB, C), lambda i: (i, 0))
    return pl.pallas_call(
        _kd_body,
        grid=(grid,),
        in_specs=[pl.BlockSpec((2, RB, 2 * Q), lambda i: (0, i, 0)),
                  nspec, nspec, nspec,
                  pl.BlockSpec(memory_space=pltpu.SMEM),
                  pl.BlockSpec((256, C), lambda i: (0, 0)),
                  pl.BlockSpec((C,), lambda i: (0,))],
        out_specs=[nspec, cspec, cspec, cspec],
        out_shape=[jax.ShapeDtypeStruct((N, 256), jnp.float32)]
        + [jax.ShapeDtypeStruct((N, C), jnp.float32)] * 3,
    )(zl2, zg, zd, zi, alpha, hw, hb)


def _ke_body(a_ref, b_ref, o_ref):
    o_ref[...] = lax.dot_general(
        a_ref[...], b_ref[...], (((1,), (1,)), ((), ())),
        preferred_element_type=jnp.float32)


def _run_ke(zg):
    cb = 1280
    return pl.pallas_call(
        _ke_body,
        grid=(N // RB, (N + cb - 1) // cb),
        in_specs=[pl.BlockSpec((RB, 256), lambda i, j: (i, 0)),
                  pl.BlockSpec((cb, 256), lambda i, j: (j, 0))],
        out_specs=pl.BlockSpec((RB, cb), lambda i, j: (i, j)),
        out_shape=jax.ShapeDtypeStruct((N, N), jnp.float32),
    )(zg, zg)


def kernel(x, adj_indices, adj_values, gcn_w, gcn_b, mean_w1, mean_b1,
           mean_w2, mean_b2, std_w1, std_b1, std_w2, std_b2,
           den_nb_w, den_nb_b, den_self_w, den_self_b, den_att_w, den_att_b,
           fus_a, fus_b, fus_alpha, head_w, head_b):
    row = adj_indices[0]
    col = adj_indices[1]
    row2d = row.reshape(NW, SB, CH)
    col2d = col.reshape(NW, SB, CH)
    val2d = adj_values.reshape(NW, SB, CH)

    rn = jax.random.uniform(jax.random.key(7), (E, 1), jnp.float32) + 1e-07
    lr2d = (jnp.log(rn) - jnp.log(1.0 - rn)).reshape(NW, SB, CH)
    ng = jax.random.normal(jax.random.key(11), (N, Z), jnp.float32)
    nd = jax.random.normal(jax.random.key(13), (N, Z), jnp.float32)

    wcat = jnp.concatenate([gcn_w, den_nb_w, den_self_w], axis=1)
    bcat = jnp.concatenate([gcn_b, den_nb_b, den_self_b], axis=0)
    a1 = den_att_w[:256].reshape(1, 256)
    a2 = den_att_w[256:].reshape(1, 256)

    xw4, s1r, s2r = _run_ka(x, wcat, bcat, a1, a2)
    xwf = xw4.reshape(4 * N, Q)
    s1 = s1r.reshape(N) + den_att_b[0]
    s2 = s2r.reshape(N)

    hg4, hd4, l0p, _nv = _run_sc1(xwf, row2d, col2d, val2d, lr2d, s1, s2)
    l0_loss = jnp.sum(l0p[:16]) / jnp.float32(E)

    zg, zd, mu, x_std, zi0, zi1 = _run_kb(
        hg4, hd4, mean_w1, mean_b1, mean_w2, mean_b2,
        std_w1, std_b1, std_w2, std_b2, ng, nd, fus_a, fus_b)

    zif = jnp.concatenate([zi0, zi1], axis=0)
    (zl2,) = _run_sc2(zif, row2d, col2d, val2d)

    zi = jnp.concatenate([zi0, zi1], axis=1)
    alpha = fus_alpha.reshape(1)

    z_fused, q_fused, q_gen, q_den = _run_kd(zl2, zg, zd, zi, alpha,
                                             head_w, head_b)

    adj_logits = _run_ke(zg)

    return (q_fused, q_gen, q_den, adj_logits, z_fused, mu, x_std, l0_loss)


# speedup vs baseline: 4.9114x; 1.6698x over previous
import functools
import math

import jax
import jax.numpy as jnp
from jax import lax
from jax.experimental import pallas as pl
from jax.experimental.pallas import tpu as pltpu
from jax.experimental.pallas import tpu_sc as plsc

N = 10000
D = 256
H = 256
Z = 256
C = 16
E = 160000
GAMMA = -0.1
ZETA = 1.1

RB = 1000
NTILES = 16
EPT = E // NTILES
CH = 80
SB = 5
NSC = EPT // (SB * CH)
NW = E // (SB * CH)
NP = 10240
RPT = NP // NTILES
Q = 64


def _ka_body(x_ref, wcat_ref, bcat_ref, a1_ref, a2_ref,
             xw4_ref, s1_ref, s2_ref):
    u = jnp.dot(x_ref[...], wcat_ref[...], preferred_element_type=jnp.float32)
    u = u + bcat_ref[...][None, :]
    xw = u[:, :256]
    t1 = jnp.maximum(u[:, 256:512], 0.0)
    t2 = jnp.maximum(u[:, 512:768], 0.0)
    for q in range(4):
        xw4_ref[q] = xw[:, q * 64:(q + 1) * 64]
    s1_ref[...] = jnp.sum(t1 * a1_ref[...], axis=1)[:, None]
    s2_ref[...] = jnp.sum(t2 * a2_ref[...], axis=1)[:, None]


def _run_ka(x, wcat, bcat, a1, a2):
    grid = N // RB
    qspec = pl.BlockSpec((4, RB, Q), lambda i: (0, i, 0))
    return pl.pallas_call(
        _ka_body,
        grid=(grid,),
        in_specs=[
            pl.BlockSpec((RB, 256), lambda i: (i, 0)),
            pl.BlockSpec((256, 768), lambda i: (0, 0)),
            pl.BlockSpec((768,), lambda i: (0,)),
            pl.BlockSpec((1, 256), lambda i: (0, 0)),
            pl.BlockSpec((1, 256), lambda i: (0, 0)),
        ],
        out_specs=[qspec,
                   pl.BlockSpec((RB, 1), lambda i: (i, 0)),
                   pl.BlockSpec((RB, 1), lambda i: (i, 0))],
        out_shape=[jax.ShapeDtypeStruct((4, N, Q), jnp.float32)]
        + [jax.ShapeDtypeStruct((N, 1), jnp.float32)] * 2,
    )(x, wcat, bcat, a1, a2)


L0C = math.log(0.1 / 1.1)


def _sigmoid16(t):
    return 1.0 / (1.0 + jnp.exp(-t))


def _sc1_body(xwf, row3d, col3d, val3d, lr3d, s1_hbm, s2_hbm,
              hg4, hd4, l0p, nv3d,
              s1v, s2v, stg_row, stg_col, stg_val, stg_nv, lrc, colq,
              rowbuf, sdb, l0v, acc_g, acc_d,
              sem_g0, sem_g1, sem_sg0, sem_sg1, sem_sd0, sem_sd1,
              sem_st0, sem_st1):
    c = lax.axis_index("c")
    s = lax.axis_index("s")
    sems_g = (sem_g0, sem_g1)
    sems_sg = (sem_sg0, sem_sg1)
    sems_sd = (sem_sd0, sem_sd1)
    sems_st = (sem_st0, sem_st1)

    pltpu.sync_copy(s1_hbm, s1v)
    pltpu.sync_copy(s2_hbm, s2v)
    l0v[...] = jnp.zeros((16,), jnp.float32)

    def gate_super(wl, _):
        w = s * NSC + wl
        pltpu.sync_copy(row3d.at[w], stg_row.at[0])
        pltpu.sync_copy(col3d.at[w], stg_col.at[0])
        pltpu.sync_copy(val3d.at[w], stg_val.at[0])
        pltpu.sync_copy(lr3d.at[w], lrc)

        def gate_chunk(m, acc):
            def gate_group(g, a):
                r16 = stg_row[0, m, pl.ds(g * 16, 16)]
                c16 = stg_col[0, m, pl.ds(g * 16, 16)]
                wgt = (plsc.load_gather(s1v, [r16])
                       + plsc.load_gather(s2v, [c16]))
                gate = _sigmoid16(lrc[m, pl.ds(g * 16, 16)] + wgt)
                msk = jnp.clip(gate * (ZETA - GAMMA) + GAMMA, 0.0, 1.0)
                stg_nv[0, m, pl.ds(g * 16, 16)] = (
                    stg_val[0, m, pl.ds(g * 16, 16)] * msk)
                return a + _sigmoid16(wgt - L0C)

            return lax.fori_loop(0, CH // 16, gate_group, acc)

        acc = lax.fori_loop(0, SB, gate_chunk, jnp.zeros((16,), jnp.float32))
        l0v[...] = l0v[...] + acc
        pltpu.sync_copy(stg_nv.at[0], nv3d.at[c * NW + w])
        return 0

    lax.fori_loop(0, NSC, gate_super, 0)
    pltpu.sync_copy(l0v, l0p.at[c * 16 + s, 0])

    for p in range(2):
        qdyn = 2 * c + p
        qoff = qdyn * N

        def issue_stage(wl_expr, slot):
            w = s * NSC + wl_expr
            pltpu.async_copy(row3d.at[w], stg_row.at[slot], sems_st[slot])
            pltpu.async_copy(col3d.at[w], stg_col.at[slot], sems_st[slot])
            pltpu.async_copy(val3d.at[w], stg_val.at[slot], sems_st[slot])
            pltpu.async_copy(nv3d.at[c * NW + w], stg_nv.at[slot],
                             sems_st[slot])

        def wait_stage(slot):
            pltpu.make_async_copy(row3d.at[s * NSC], stg_row.at[slot],
                                  sems_st[slot]).wait()
            pltpu.make_async_copy(col3d.at[s * NSC], stg_col.at[slot],
                                  sems_st[slot]).wait()
            pltpu.make_async_copy(val3d.at[s * NSC], stg_val.at[slot],
                                  sems_st[slot]).wait()
            pltpu.make_async_copy(nv3d.at[0], stg_nv.at[slot],
                                  sems_st[slot]).wait()

        def emit_chunk(mi, slot, gp, first_pred, nxt):
            np_ = 1 - gp
            if nxt is not None:
                nmi, nslot, stale = nxt
                if stale:
                    wait_stage(nslot)

                def addq(g, _):
                    colq[np_, pl.ds(g * 16, 16)] = (
                        stg_col[nslot, nmi, pl.ds(g * 16, 16)] + qoff)
                    return 0

                lax.fori_loop(0, CH // 16, addq, 0)

            def wait_prev_scatters():
                pltpu.make_async_copy(rowbuf.at[np_],
                                      acc_g.at[colq.at[np_]],
                                      sems_sg[np_]).wait()
                pltpu.make_async_copy(sdb.at[np_],
                                      acc_d.at[colq.at[np_]],
                                      sems_sd[np_]).wait()

            if first_pred is None:
                wait_prev_scatters()
            else:
                pl.when(first_pred)(wait_prev_scatters)

            if nxt is not None:
                pltpu.async_copy(xwf.at[colq.at[np_]], rowbuf.at[np_],
                                 sems_g[np_])
            pltpu.make_async_copy(xwf.at[colq.at[gp]], rowbuf.at[gp],
                                  sems_g[gp]).wait()

            def scale(k, _):
                i16s = jnp.full((16,), slot, jnp.int32)
                i16m = jnp.full((16,), mi, jnp.int32)
                i16k = jnp.full((16,), k, jnp.int32)
                vg = plsc.load_gather(stg_val, [i16s, i16m, i16k])
                vd = plsc.load_gather(stg_nv, [i16s, i16m, i16k])
                for u in range(Q // 16):
                    r = rowbuf[gp, k, pl.ds(u * 16, 16)]
                    sdb[gp, k, pl.ds(u * 16, 16)] = r * vd
                    rowbuf[gp, k, pl.ds(u * 16, 16)] = r * vg
                return 0

            lax.fori_loop(0, CH, scale, 0)
            pltpu.async_copy(rowbuf.at[gp], acc_g.at[stg_row.at[slot, mi]],
                             sems_sg[gp], add=True)
            pltpu.async_copy(sdb.at[gp], acc_d.at[stg_row.at[slot, mi]],
                             sems_sd[gp], add=True)

        def zrow(i, _):
            for u in range(Q // 16):
                rowbuf[0, i, pl.ds(u * 16, 16)] = jnp.zeros((16,), jnp.float32)
            return 0

        lax.fori_loop(0, CH, zrow, 0)
        for i in range(RPT // CH):
            pltpu.sync_copy(rowbuf.at[0], acc_g.at[pl.ds(s * RPT + i * CH, CH)])
            pltpu.sync_copy(rowbuf.at[0], acc_d.at[pl.ds(s * RPT + i * CH, CH)])
        plsc.subcore_barrier()

        pltpu.sync_copy(row3d.at[s * NSC], stg_row.at[0])
        pltpu.sync_copy(col3d.at[s * NSC], stg_col.at[0])
        pltpu.sync_copy(val3d.at[s * NSC], stg_val.at[0])
        pltpu.sync_copy(nv3d.at[c * NW + s * NSC], stg_nv.at[0])

        def addq0(g, _):
            colq[0, pl.ds(g * 16, 16)] = (
                stg_col[0, 0, pl.ds(g * 16, 16)] + qoff)
            return 0

        lax.fori_loop(0, CH // 16, addq0, 0)
        pltpu.async_copy(xwf.at[colq.at[0]], rowbuf.at[0], sems_g[0])

        def super_pair(t, _):
            emit_chunk(0, 0, 0, t > 0, (1, 0, False))
            issue_stage(2 * t + 1, 1)
            for mi in range(1, 4):
                emit_chunk(mi, 0, mi & 1, None, (mi + 1, 0, False))
            emit_chunk(4, 0, 0, None, (0, 1, True))
            emit_chunk(0, 1, 1, None, (1, 1, False))
            issue_stage(2 * t + 2, 0)
            for mi in range(1, 4):
                emit_chunk(mi, 1, (1 + mi) & 1, None, (mi + 1, 1, False))
            emit_chunk(4, 1, 1, None, (0, 0, True))
            return 0

        lax.fori_loop(0, (NSC - 1) // 2, super_pair, 0)

        for mi in range(4):
            emit_chunk(mi, 0, mi & 1, None, (mi + 1, 0, False))
        emit_chunk(4, 0, 0, None, None)
        pltpu.make_async_copy(rowbuf.at[0], acc_g.at[colq.at[0]],
                              sems_sg[0]).wait()
        pltpu.make_async_copy(sdb.at[0], acc_d.at[colq.at[0]],
                              sems_sd[0]).wait()
        plsc.subcore_barrier()

        for i in range(RPT // CH):
            r0 = s * RPT + i * CH
            pltpu.sync_copy(acc_g.at[pl.ds(r0, CH)], rowbuf.at[0])
            pltpu.sync_copy(rowbuf.at[0], hg4.at[qdyn, pl.ds(r0, CH)])
            pltpu.sync_copy(acc_d.at[pl.ds(r0, CH)], sdb.at[0])
            pltpu.sync_copy(sdb.at[0], hd4.at[qdyn, pl.ds(r0, CH)])
        if p == 0:
            plsc.subcore_barrier()


def _run_sc1(xwf, row3d, col3d, val3d, lr3d, s1, s2):
    mesh = plsc.VectorSubcoreMesh(core_axis_name="c", subcore_axis_name="s")
    f = pl.kernel(
        _sc1_body,
        compiler_params=pltpu.CompilerParams(
            use_tc_tiling_on_sc=False, needs_layout_passes=False),
        out_type=[
            jax.ShapeDtypeStruct((4, NP, Q), jnp.float32),
            jax.ShapeDtypeStruct((4, NP, Q), jnp.float32),
            jax.ShapeDtypeStruct((32, 1, 16), jnp.float32),
            jax.ShapeDtypeStruct((2 * NW, SB, CH), jnp.float32),
        ],
        mesh=mesh,
        scratch_types=[
            pltpu.VMEM((N,), jnp.float32),
            pltpu.VMEM((N,), jnp.float32),
            pltpu.VMEM((2, SB, CH), jnp.int32),
            pltpu.VMEM((2, SB, CH), jnp.int32),
            pltpu.VMEM((2, SB, CH), jnp.float32),
            pltpu.VMEM((2, SB, CH), jnp.float32),
            pltpu.VMEM((SB, CH), jnp.float32),
            pltpu.VMEM((2, CH), jnp.int32),
            pltpu.VMEM((2, CH, Q), jnp.float32),
            pltpu.VMEM((2, CH, Q), jnp.float32),
            pltpu.VMEM((16,), jnp.float32),
            pltpu.VMEM_SHARED((NP, Q), jnp.float32),
            pltpu.VMEM_SHARED((NP, Q), jnp.float32),
            pltpu.SemaphoreType.DMA,
            pltpu.SemaphoreType.DMA,
            pltpu.SemaphoreType.DMA,
            pltpu.SemaphoreType.DMA,
            pltpu.SemaphoreType.DMA,
            pltpu.SemaphoreType.DMA,
            pltpu.SemaphoreType.DMA,
            pltpu.SemaphoreType.DMA,
        ],
    )
    return f(xwf, row3d, col3d, val3d, lr3d, s1, s2)


def _sc2_body(zif, row3d, col3d, val3d,
              zl2,
              stg_row, stg_col, stg_val, colq, rowbuf, acc,
              sem_g0, sem_g1, sem_sg0, sem_sg1, sem_st0, sem_st1):
    c = lax.axis_index("c")
    s = lax.axis_index("s")
    QH = 2 * Q
    sems_g = (sem_g0, sem_g1)
    sems_sg = (sem_sg0, sem_sg1)
    sems_st = (sem_st0, sem_st1)
    qoff = c * N

    def issue_stage(wl_expr, slot):
        w = s * NSC + wl_expr
        pltpu.async_copy(row3d.at[w], stg_row.at[slot], sems_st[slot])
        pltpu.async_copy(col3d.at[w], stg_col.at[slot], sems_st[slot])
        pltpu.async_copy(val3d.at[w], stg_val.at[slot], sems_st[slot])

    def wait_stage(slot):
        pltpu.make_async_copy(row3d.at[s * NSC], stg_row.at[slot],
                              sems_st[slot]).wait()
        pltpu.make_async_copy(col3d.at[s * NSC], stg_col.at[slot],
                              sems_st[slot]).wait()
        pltpu.make_async_copy(val3d.at[s * NSC], stg_val.at[slot],
                              sems_st[slot]).wait()

    def emit_chunk(mi, slot, gp, first_pred, nxt):
        np_ = 1 - gp
        if nxt is not None:
            nmi, nslot, stale = nxt
            if stale:
                wait_stage(nslot)

            def addq(g, _):
                colq[np_, pl.ds(g * 16, 16)] = (
                    stg_col[nslot, nmi, pl.ds(g * 16, 16)] + qoff)
                return 0

            lax.fori_loop(0, CH // 16, addq, 0)

        def wait_prev_scatter():
            pltpu.make_async_copy(rowbuf.at[np_], acc.at[colq.at[np_]],
                                  sems_sg[np_]).wait()

        if first_pred is None:
            wait_prev_scatter()
        else:
            pl.when(first_pred)(wait_prev_scatter)

        if nxt is not None:
            pltpu.async_copy(zif.at[colq.at[np_]], rowbuf.at[np_],
                             sems_g[np_])
        pltpu.make_async_copy(zif.at[colq.at[gp]], rowbuf.at[gp],
                              sems_g[gp]).wait()

        def scale(k, _):
            i16s = jnp.full((16,), slot, jnp.int32)
            i16m = jnp.full((16,), mi, jnp.int32)
            i16k = jnp.full((16,), k, jnp.int32)
            vg = plsc.load_gather(stg_val, [i16s, i16m, i16k])
            for u in range(QH // 16):
                rowbuf[gp, k, pl.ds(u * 16, 16)] = (
                    rowbuf[gp, k, pl.ds(u * 16, 16)] * vg)
            return 0

        lax.fori_loop(0, CH, scale, 0)
        pltpu.async_copy(rowbuf.at[gp], acc.at[stg_row.at[slot, mi]],
                         sems_sg[gp], add=True)

    def zrow(i, _):
        for u in range(QH // 16):
            rowbuf[0, i, pl.ds(u * 16, 16)] = jnp.zeros((16,), jnp.float32)
        return 0

    lax.fori_loop(0, CH, zrow, 0)
    for i in range(RPT // CH):
        pltpu.sync_copy(rowbuf.at[0], acc.at[pl.ds(s * RPT + i * CH, CH)])
    plsc.subcore_barrier()

    pltpu.sync_copy(row3d.at[s * NSC], stg_row.at[0])
    pltpu.sync_copy(col3d.at[s * NSC], stg_col.at[0])
    pltpu.sync_copy(val3d.at[s * NSC], stg_val.at[0])

    def addq0(g, _):
        colq[0, pl.ds(g * 16, 16)] = stg_col[0, 0, pl.ds(g * 16, 16)] + qoff
        return 0

    lax.fori_loop(0, CH // 16, addq0, 0)
    pltpu.async_copy(zif.at[colq.at[0]], rowbuf.at[0], sems_g[0])

    def super_pair(t, _):
        emit_chunk(0, 0, 0, t > 0, (1, 0, False))
        issue_stage(2 * t + 1, 1)
        for mi in range(1, 4):
            emit_chunk(mi, 0, mi & 1, None, (mi + 1, 0, False))
        emit_chunk(4, 0, 0, None, (0, 1, True))
        emit_chunk(0, 1, 1, None, (1, 1, False))
        issue_stage(2 * t + 2, 0)
        for mi in range(1, 4):
            emit_chunk(mi, 1, (1 + mi) & 1, None, (mi + 1, 1, False))
        emit_chunk(4, 1, 1, None, (0, 0, True))
        return 0

    lax.fori_loop(0, (NSC - 1) // 2, super_pair, 0)

    for mi in range(4):
        emit_chunk(mi, 0, mi & 1, None, (mi + 1, 0, False))
    emit_chunk(4, 0, 0, None, None)
    pltpu.make_async_copy(rowbuf.at[0], acc.at[colq.at[0]],
                          sems_sg[0]).wait()
    plsc.subcore_barrier()

    for i in range(RPT // CH):
        r0 = s * RPT + i * CH
        pltpu.sync_copy(acc.at[pl.ds(r0, CH)], rowbuf.at[0])
        pltpu.sync_copy(rowbuf.at[0], zl2.at[c, pl.ds(r0, CH)])


def _run_sc2(zif, row3d, col3d, val3d):
    mesh = plsc.VectorSubcoreMesh(core_axis_name="c", subcore_axis_name="s")
    f = pl.kernel(
        _sc2_body,
        compiler_params=pltpu.CompilerParams(
            use_tc_tiling_on_sc=False, needs_layout_passes=False),
        out_type=[jax.ShapeDtypeStruct((2, NP, 2 * Q), jnp.float32)],
        mesh=mesh,
        scratch_types=[
            pltpu.VMEM((2, SB, CH), jnp.int32),
            pltpu.VMEM((2, SB, CH), jnp.int32),
            pltpu.VMEM((2, SB, CH), jnp.float32),
            pltpu.VMEM((2, CH), jnp.int32),
            pltpu.VMEM((2, CH, 2 * Q), jnp.float32),
            pltpu.VMEM_SHARED((NP, 2 * Q), jnp.float32),
            pltpu.SemaphoreType.DMA,
            pltpu.SemaphoreType.DMA,
            pltpu.SemaphoreType.DMA,
            pltpu.SemaphoreType.DMA,
            pltpu.SemaphoreType.DMA,
            pltpu.SemaphoreType.DMA,
        ],
    )
    return f(zif, row3d, col3d, val3d)


def _softplus(t):
    return jnp.maximum(t, 0.0) + jnp.log1p(jnp.exp(-jnp.abs(t)))


def _kb_body(hg_ref, hd_ref, mw1_ref, mb1_ref, mw2_ref, mb2_ref,
             sw1_ref, sb1_ref, sw2_ref, sb2_ref, ng_ref, nd_ref,
             fa_ref, fb_ref,
             zg_ref, zd_ref, mu_ref, std_ref,
             zi0_ref, zi1_ref):
    def enc(h):
        m1 = jnp.maximum(
            jnp.dot(h, mw1_ref[...], preferred_element_type=jnp.float32)
            + mb1_ref[...][None, :], 0.0)
        mean = (jnp.dot(m1, mw2_ref[...], preferred_element_type=jnp.float32)
                + mb2_ref[...][None, :])
        s1 = jnp.maximum(
            jnp.dot(h, sw1_ref[...], preferred_element_type=jnp.float32)
            + sb1_ref[...][None, :], 0.0)
        std = _softplus(
            jnp.dot(s1, sw2_ref[...], preferred_element_type=jnp.float32)
            + sb2_ref[...][None, :])
        return mean, std

    hg = jnp.concatenate(
        [jnp.maximum(hg_ref[q], 0.0) for q in range(4)], axis=1)
    hd = jnp.concatenate(
        [jnp.maximum(hd_ref[q], 0.0) for q in range(4)], axis=1)
    mu, std_g = enc(hg)
    mu_d, std_d = enc(hd)
    zg = ng_ref[...] * std_g + mu
    zd = nd_ref[...] * std_d + mu_d
    zi = fa_ref[...] * zg + fb_ref[...] * zd
    zg_ref[...] = zg
    zd_ref[...] = zd
    mu_ref[...] = mu
    std_ref[...] = std_g
    zi0_ref[...] = zi[:, 0:128]
    zi1_ref[...] = zi[:, 128:256]


def _run_kb(hg4, hd4, mw1, mb1, mw2, mb2, sw1, sb1, sw2, sb2, ng, nd, fa, fb):
    grid = N // RB
    h4spec = pl.BlockSpec((4, RB, Q), lambda i: (0, i, 0))
    wspec = pl.BlockSpec((256, 256), lambda i: (0, 0))
    bspec = pl.BlockSpec((256,), lambda i: (0,))
    nspec = pl.BlockSpec((RB, 256), lambda i: (i, 0))
    hspec = pl.BlockSpec((RB, 128), lambda i: (i, 0))
    return pl.pallas_call(
        _kb_body,
        grid=(grid,),
        in_specs=[h4spec, h4spec, wspec, bspec, wspec, bspec,
                  wspec, bspec, wspec, bspec, nspec, nspec, nspec, nspec],
        out_specs=[nspec, nspec, nspec, nspec, hspec, hspec],
        out_shape=[jax.ShapeDtypeStruct((N, 256), jnp.float32)] * 4
        + [jax.ShapeDtypeStruct((N, 128), jnp.float32)] * 2,
    )(hg4, hd4, mw1, mb1, mw2, mb2, sw1, sb1, sw2, sb2, ng, nd, fa, fb)


def _softmax(t):
    m = jnp.max(t, axis=1, keepdims=True)
    e = jnp.exp(t - m)
    return e / jnp.sum(e, axis=1, keepdims=True)


def _kd_body(zl2_ref, zg_ref, zd_ref, zi_ref, alpha_ref, hw_ref, hb_ref,
             zf_ref, qf_ref, qg_ref, qd_ref):
    zl = jnp.concatenate([zl2_ref[q] for q in range(2)], axis=1)
    alpha = alpha_ref[0]
    zi = zi_ref[...]
    zf = alpha * zl + (1.0 - alpha) * zi
    zf_ref[...] = zf
    hw = hw_ref[...]
    hb = hb_ref[...][None, :]
    qf_ref[...] = _softmax(
        jnp.dot(zf, hw, preferred_element_type=jnp.float32) + hb)
    qg_ref[...] = _softmax(
        jnp.dot(zg_ref[...], hw, preferred_element_type=jnp.float32) + hb)
    qd_ref[...] = _softmax(
        jnp.dot(zd_ref[...], hw, preferred_element_type=jnp.float32) + hb)


def _run_kd(zl2, zg, zd, zi, alpha, hw, hb):
    grid = N // RB
    nspec = pl.BlockSpec((RB, 256), lambda i: (i, 0))
    cspec = pl.BlockSpec((RB, C), lambda i: (i, 0))
    return pl.pallas_call(
        _kd_body,
        grid=(grid,),
        in_specs=[pl.BlockSpec((2, RB, 2 * Q), lambda i: (0, i, 0)),
                  nspec, nspec, nspec,
                  pl.BlockSpec(memory_space=pltpu.SMEM),
                  pl.BlockSpec((256, C), lambda i: (0, 0)),
                  pl.BlockSpec((C,), lambda i: (0,))],
        out_specs=[nspec, cspec, cspec, cspec],
        out_shape=[jax.ShapeDtypeStruct((N, 256), jnp.float32)]
        + [jax.ShapeDtypeStruct((N, C), jnp.float32)] * 3,
    )(zl2, zg, zd, zi, alpha, hw, hb)


def _ke_body(a_ref, b_ref, o_ref):
    o_ref[...] = lax.dot_general(
        a_ref[...], b_ref[...], (((1,), (1,)), ((), ())),
        preferred_element_type=jnp.float32)


def _run_ke(zg):
    cb = 1280
    return pl.pallas_call(
        _ke_body,
        grid=(N // RB, (N + cb - 1) // cb),
        in_specs=[pl.BlockSpec((RB, 256), lambda i, j: (i, 0)),
                  pl.BlockSpec((cb, 256), lambda i, j: (j, 0))],
        out_specs=pl.BlockSpec((RB, cb), lambda i, j: (i, j)),
        out_shape=jax.ShapeDtypeStruct((N, N), jnp.float32),
    )(zg, zg)


def kernel(x, adj_indices, adj_values, gcn_w, gcn_b, mean_w1, mean_b1,
           mean_w2, mean_b2, std_w1, std_b1, std_w2, std_b2,
           den_nb_w, den_nb_b, den_self_w, den_self_b, den_att_w, den_att_b,
           fus_a, fus_b, fus_alpha, head_w, head_b):
    row = adj_indices[0]
    col = adj_indices[1]
    row2d = row.reshape(NW, SB, CH)
    col2d = col.reshape(NW, SB, CH)
    val2d = adj_values.reshape(NW, SB, CH)

    rn = jax.random.uniform(jax.random.key(7), (E, 1), jnp.float32) + 1e-07
    lr2d = (jnp.log(rn) - jnp.log(1.0 - rn)).reshape(NW, SB, CH)
    ng = jax.random.normal(jax.random.key(11), (N, Z), jnp.float32)
    nd = jax.random.normal(jax.random.key(13), (N, Z), jnp.float32)

    wcat = jnp.concatenate([gcn_w, den_nb_w, den_self_w], axis=1)
    bcat = jnp.concatenate([gcn_b, den_nb_b, den_self_b], axis=0)
    a1 = den_att_w[:256].reshape(1, 256)
    a2 = den_att_w[256:].reshape(1, 256)

    xw4, s1r, s2r = _run_ka(x, wcat, bcat, a1, a2)
    xwf = xw4.reshape(4 * N, Q)
    s1 = s1r.reshape(N) + den_att_b[0]
    s2 = s2r.reshape(N)

    hg4, hd4, l0p, _nv = _run_sc1(xwf, row2d, col2d, val2d, lr2d, s1, s2)
    l0_loss = jnp.sum(l0p[:16]) / jnp.float32(E)

    zg, zd, mu, x_std, zi0, zi1 = _run_kb(
        hg4, hd4, mean_w1, mean_b1, mean_w2, mean_b2,
        std_w1, std_b1, std_w2, std_b2, ng, nd, fus_a, fus_b)

    zif = jnp.concatenate([zi0, zi1], axis=0)
    (zl2,) = _run_sc2(zif, row2d, col2d, val2d)

    zi = jnp.concatenate([zi0, zi1], axis=1)
    alpha = fus_alpha.reshape(1)

    z_fused, q_fused, q_gen, q_den = _run_kd(zl2, zg, zd, zi, alpha,
                                             head_w, head_b)

    adj_logits = _run_ke(zg)

    return (q_fused, q_gen, q_den, adj_logits, z_fused, mu, x_std, l0_loss)


# speedup vs baseline: 5.1588x; 1.0504x over previous
import functools
import math

import jax
import jax.numpy as jnp
from jax import lax
from jax.experimental import pallas as pl
from jax.experimental.pallas import tpu as pltpu
from jax.experimental.pallas import tpu_sc as plsc

N = 10000
D = 256
H = 256
Z = 256
C = 16
E = 160000
GAMMA = -0.1
ZETA = 1.1

RB = 1000
NTILES = 16
EPT = E // NTILES
CH = 80
SB = 5
NSC = EPT // (SB * CH)
NW = E // (SB * CH)
NP = 10240
RPT = NP // NTILES
Q = 64


def _ka_body(x_ref, wcat_ref, bcat_ref, a1_ref, a2_ref,
             xw4_ref, s1_ref, s2_ref):
    u = jnp.dot(x_ref[...], wcat_ref[...], preferred_element_type=jnp.float32)
    u = u + bcat_ref[...][None, :]
    xw = u[:, :256]
    t1 = jnp.maximum(u[:, 256:512], 0.0)
    t2 = jnp.maximum(u[:, 512:768], 0.0)
    for q in range(4):
        xw4_ref[q] = xw[:, q * 64:(q + 1) * 64]
    s1_ref[...] = jnp.sum(t1 * a1_ref[...], axis=1)[:, None]
    s2_ref[...] = jnp.sum(t2 * a2_ref[...], axis=1)[:, None]


def _run_ka(x, wcat, bcat, a1, a2):
    grid = N // RB
    qspec = pl.BlockSpec((4, RB, Q), lambda i: (0, i, 0))
    return pl.pallas_call(
        _ka_body,
        grid=(grid,),
        in_specs=[
            pl.BlockSpec((RB, 256), lambda i: (i, 0)),
            pl.BlockSpec((256, 768), lambda i: (0, 0)),
            pl.BlockSpec((768,), lambda i: (0,)),
            pl.BlockSpec((1, 256), lambda i: (0, 0)),
            pl.BlockSpec((1, 256), lambda i: (0, 0)),
        ],
        out_specs=[qspec,
                   pl.BlockSpec((RB, 1), lambda i: (i, 0)),
                   pl.BlockSpec((RB, 1), lambda i: (i, 0))],
        out_shape=[jax.ShapeDtypeStruct((4, N, Q), jnp.float32)]
        + [jax.ShapeDtypeStruct((N, 1), jnp.float32)] * 2,
    )(x, wcat, bcat, a1, a2)


L0C = math.log(0.1 / 1.1)


def _sigmoid16(t):
    return 1.0 / (1.0 + jnp.exp(-t))


def _sc1_body(xwf, row3d, col3d, val3d, lr3d, s1_hbm, s2_hbm,
              hg4, hd4, l0p, nv3d,
              s1v, s2v, stg_row, stg_col, stg_val, stg_nv, stg_lr, colq,
              rowbuf, sdb, l0v, acc_g, acc_d,
              sem_g0, sem_g1, sem_sg0, sem_sg1, sem_sd0, sem_sd1,
              sem_st0, sem_st1, sem_nv0, sem_nv1):
    c = lax.axis_index("c")
    s = lax.axis_index("s")
    sems_g = (sem_g0, sem_g1)
    sems_sg = (sem_sg0, sem_sg1)
    sems_sd = (sem_sd0, sem_sd1)
    sems_st = (sem_st0, sem_st1)
    sems_nv = (sem_nv0, sem_nv1)

    pltpu.sync_copy(s1_hbm, s1v)
    pltpu.sync_copy(s2_hbm, s2v)
    l0v[...] = jnp.zeros((16,), jnp.float32)

    def gate_chunk(nslot, nmi):
        def gate_group(g, a):
            r16 = stg_row[nslot, nmi, pl.ds(g * 16, 16)]
            c16 = stg_col[nslot, nmi, pl.ds(g * 16, 16)]
            wgt = (plsc.load_gather(s1v, [r16])
                   + plsc.load_gather(s2v, [c16]))
            gate = _sigmoid16(stg_lr[nslot, nmi, pl.ds(g * 16, 16)] + wgt)
            msk = jnp.clip(gate * (ZETA - GAMMA) + GAMMA, 0.0, 1.0)
            stg_nv[nslot, nmi, pl.ds(g * 16, 16)] = (
                stg_val[nslot, nmi, pl.ds(g * 16, 16)] * msk)
            return a + _sigmoid16(wgt - L0C)

        acc = lax.fori_loop(0, CH // 16, gate_group,
                            jnp.zeros((16,), jnp.float32))
        l0v[...] = l0v[...] + acc

    for p in range(2):
        qdyn = 2 * c + p
        qoff = qdyn * N

        def issue_stage(wl_expr, slot):
            w = s * NSC + wl_expr
            pltpu.async_copy(row3d.at[w], stg_row.at[slot], sems_st[slot])
            pltpu.async_copy(col3d.at[w], stg_col.at[slot], sems_st[slot])
            pltpu.async_copy(val3d.at[w], stg_val.at[slot], sems_st[slot])
            if p == 0:
                pltpu.async_copy(lr3d.at[w], stg_lr.at[slot], sems_st[slot])
            else:
                pltpu.async_copy(nv3d.at[c * NW + w], stg_nv.at[slot],
                                 sems_st[slot])

        def wait_stage(slot):
            pltpu.make_async_copy(row3d.at[s * NSC], stg_row.at[slot],
                                  sems_st[slot]).wait()
            pltpu.make_async_copy(col3d.at[s * NSC], stg_col.at[slot],
                                  sems_st[slot]).wait()
            pltpu.make_async_copy(val3d.at[s * NSC], stg_val.at[slot],
                                  sems_st[slot]).wait()
            if p == 0:
                pltpu.make_async_copy(lr3d.at[s * NSC], stg_lr.at[slot],
                                      sems_st[slot]).wait()
            else:
                pltpu.make_async_copy(nv3d.at[0], stg_nv.at[slot],
                                      sems_st[slot]).wait()

        def emit_chunk(mi, slot, gp, first_pred, nxt, nv_wait_pred=False):
            np_ = 1 - gp
            if nxt is not None:
                nmi, nslot, stale = nxt
                if stale:
                    wait_stage(nslot)

                def addq(g, _):
                    colq[np_, pl.ds(g * 16, 16)] = (
                        stg_col[nslot, nmi, pl.ds(g * 16, 16)] + qoff)
                    return 0

                lax.fori_loop(0, CH // 16, addq, 0)

            def wait_prev_scatters():
                pltpu.make_async_copy(rowbuf.at[np_],
                                      acc_g.at[colq.at[np_]],
                                      sems_sg[np_]).wait()
                pltpu.make_async_copy(sdb.at[np_],
                                      acc_d.at[colq.at[np_]],
                                      sems_sd[np_]).wait()

            if first_pred is None:
                wait_prev_scatters()
            else:
                pl.when(first_pred)(wait_prev_scatters)

            if nxt is not None:
                pltpu.async_copy(xwf.at[colq.at[np_]], rowbuf.at[np_],
                                 sems_g[np_])
                if p == 0:
                    nmi, nslot, stale = nxt
                    if nv_wait_pred is True:
                        pltpu.make_async_copy(stg_nv.at[nslot], nv3d.at[0],
                                              sems_nv[nslot]).wait()
                    elif nv_wait_pred is not False:
                        @pl.when(nv_wait_pred)
                        def _():
                            pltpu.make_async_copy(stg_nv.at[nslot],
                                                  nv3d.at[0],
                                                  sems_nv[nslot]).wait()
                    gate_chunk(nslot, nmi)
            pltpu.make_async_copy(xwf.at[colq.at[gp]], rowbuf.at[gp],
                                  sems_g[gp]).wait()

            def scale(k, _):
                i16s = jnp.full((16,), slot, jnp.int32)
                i16m = jnp.full((16,), mi, jnp.int32)
                i16k = jnp.full((16,), k, jnp.int32)
                vg = plsc.load_gather(stg_val, [i16s, i16m, i16k])
                vd = plsc.load_gather(stg_nv, [i16s, i16m, i16k])
                for u in range(Q // 16):
                    r = rowbuf[gp, k, pl.ds(u * 16, 16)]
                    sdb[gp, k, pl.ds(u * 16, 16)] = r * vd
                    rowbuf[gp, k, pl.ds(u * 16, 16)] = r * vg
                return 0

            lax.fori_loop(0, CH, scale, 0)
            pltpu.async_copy(rowbuf.at[gp], acc_g.at[stg_row.at[slot, mi]],
                             sems_sg[gp], add=True)
            pltpu.async_copy(sdb.at[gp], acc_d.at[stg_row.at[slot, mi]],
                             sems_sd[gp], add=True)

        def flush_nv(wl_expr, slot):
            if p == 0:
                pltpu.async_copy(stg_nv.at[slot],
                                 nv3d.at[c * NW + s * NSC + wl_expr],
                                 sems_nv[slot])

        def zrow(i, _):
            for u in range(Q // 16):
                rowbuf[0, i, pl.ds(u * 16, 16)] = jnp.zeros((16,), jnp.float32)
            return 0

        lax.fori_loop(0, CH, zrow, 0)
        for i in range(RPT // CH):
            pltpu.sync_copy(rowbuf.at[0], acc_g.at[pl.ds(s * RPT + i * CH, CH)])
            pltpu.sync_copy(rowbuf.at[0], acc_d.at[pl.ds(s * RPT + i * CH, CH)])
        plsc.subcore_barrier()

        pltpu.sync_copy(row3d.at[s * NSC], stg_row.at[0])
        pltpu.sync_copy(col3d.at[s * NSC], stg_col.at[0])
        pltpu.sync_copy(val3d.at[s * NSC], stg_val.at[0])
        if p == 0:
            pltpu.sync_copy(lr3d.at[s * NSC], stg_lr.at[0])
            gate_chunk(0, 0)
        else:
            pltpu.sync_copy(nv3d.at[c * NW + s * NSC], stg_nv.at[0])

        def addq0(g, _):
            colq[0, pl.ds(g * 16, 16)] = (
                stg_col[0, 0, pl.ds(g * 16, 16)] + qoff)
            return 0

        lax.fori_loop(0, CH // 16, addq0, 0)
        pltpu.async_copy(xwf.at[colq.at[0]], rowbuf.at[0], sems_g[0])

        def super_pair(t, _):
            emit_chunk(0, 0, 0, t > 0, (1, 0, False))
            issue_stage(2 * t + 1, 1)
            for mi in range(1, 4):
                emit_chunk(mi, 0, mi & 1, None, (mi + 1, 0, False))
            emit_chunk(4, 0, 0, None, (0, 1, True), nv_wait_pred=t > 0)
            flush_nv(2 * t, 0)
            emit_chunk(0, 1, 1, None, (1, 1, False))
            issue_stage(2 * t + 2, 0)
            for mi in range(1, 4):
                emit_chunk(mi, 1, (1 + mi) & 1, None, (mi + 1, 1, False))
            emit_chunk(4, 1, 1, None, (0, 0, True), nv_wait_pred=True)
            flush_nv(2 * t + 1, 1)
            return 0

        lax.fori_loop(0, (NSC - 1) // 2, super_pair, 0)

        for mi in range(4):
            emit_chunk(mi, 0, mi & 1, None, (mi + 1, 0, False))
        emit_chunk(4, 0, 0, None, None)
        flush_nv(24, 0)
        pltpu.make_async_copy(rowbuf.at[0], acc_g.at[colq.at[0]],
                              sems_sg[0]).wait()
        pltpu.make_async_copy(sdb.at[0], acc_d.at[colq.at[0]],
                              sems_sd[0]).wait()
        if p == 0:
            pltpu.make_async_copy(stg_nv.at[0], nv3d.at[0], sems_nv[0]).wait()
            pltpu.make_async_copy(stg_nv.at[1], nv3d.at[0], sems_nv[1]).wait()
        plsc.subcore_barrier()

        for i in range(RPT // CH):
            r0 = s * RPT + i * CH
            pltpu.sync_copy(acc_g.at[pl.ds(r0, CH)], rowbuf.at[0])
            pltpu.sync_copy(rowbuf.at[0], hg4.at[qdyn, pl.ds(r0, CH)])
            pltpu.sync_copy(acc_d.at[pl.ds(r0, CH)], sdb.at[0])
            pltpu.sync_copy(sdb.at[0], hd4.at[qdyn, pl.ds(r0, CH)])
        if p == 0:
            plsc.subcore_barrier()

    pltpu.sync_copy(l0v, l0p.at[c * 16 + s, 0])


def _run_sc1(xwf, row3d, col3d, val3d, lr3d, s1, s2):
    mesh = plsc.VectorSubcoreMesh(core_axis_name="c", subcore_axis_name="s")
    f = pl.kernel(
        _sc1_body,
        compiler_params=pltpu.CompilerParams(
            use_tc_tiling_on_sc=False, needs_layout_passes=False),
        out_type=[
            jax.ShapeDtypeStruct((4, NP, Q), jnp.float32),
            jax.ShapeDtypeStruct((4, NP, Q), jnp.float32),
            jax.ShapeDtypeStruct((32, 1, 16), jnp.float32),
            jax.ShapeDtypeStruct((2 * NW, SB, CH), jnp.float32),
        ],
        mesh=mesh,
        scratch_types=[
            pltpu.VMEM((N,), jnp.float32),
            pltpu.VMEM((N,), jnp.float32),
            pltpu.VMEM((2, SB, CH), jnp.int32),
            pltpu.VMEM((2, SB, CH), jnp.int32),
            pltpu.VMEM((2, SB, CH), jnp.float32),
            pltpu.VMEM((2, SB, CH), jnp.float32),
            pltpu.VMEM((2, SB, CH), jnp.float32),
            pltpu.VMEM((2, CH), jnp.int32),
            pltpu.VMEM((2, CH, Q), jnp.float32),
            pltpu.VMEM((2, CH, Q), jnp.float32),
            pltpu.VMEM((16,), jnp.float32),
            pltpu.VMEM_SHARED((NP, Q), jnp.float32),
            pltpu.VMEM_SHARED((NP, Q), jnp.float32),
            pltpu.SemaphoreType.DMA,
            pltpu.SemaphoreType.DMA,
            pltpu.SemaphoreType.DMA,
            pltpu.SemaphoreType.DMA,
            pltpu.SemaphoreType.DMA,
            pltpu.SemaphoreType.DMA,
            pltpu.SemaphoreType.DMA,
            pltpu.SemaphoreType.DMA,
            pltpu.SemaphoreType.DMA,
            pltpu.SemaphoreType.DMA,
        ],
    )
    return f(xwf, row3d, col3d, val3d, lr3d, s1, s2)


def _sc2_body(zif, row3d, col3d, val3d,
              zl2,
              stg_row, stg_col, stg_val, colq, rowbuf, acc,
              sem_g0, sem_g1, sem_sg0, sem_sg1, sem_st0, sem_st1):
    c = lax.axis_index("c")
    s = lax.axis_index("s")
    QH = 2 * Q
    sems_g = (sem_g0, sem_g1)
    sems_sg = (sem_sg0, sem_sg1)
    sems_st = (sem_st0, sem_st1)
    qoff = c * N

    def issue_stage(wl_expr, slot):
        w = s * NSC + wl_expr
        pltpu.async_copy(row3d.at[w], stg_row.at[slot], sems_st[slot])
        pltpu.async_copy(col3d.at[w], stg_col.at[slot], sems_st[slot])
        pltpu.async_copy(val3d.at[w], stg_val.at[slot], sems_st[slot])

    def wait_stage(slot):
        pltpu.make_async_copy(row3d.at[s * NSC], stg_row.at[slot],
                              sems_st[slot]).wait()
        pltpu.make_async_copy(col3d.at[s * NSC], stg_col.at[slot],
                              sems_st[slot]).wait()
        pltpu.make_async_copy(val3d.at[s * NSC], stg_val.at[slot],
                              sems_st[slot]).wait()

    def emit_chunk(mi, slot, gp, first_pred, nxt):
        np_ = 1 - gp
        if nxt is not None:
            nmi, nslot, stale = nxt
            if stale:
                wait_stage(nslot)

            def addq(g, _):
                colq[np_, pl.ds(g * 16, 16)] = (
                    stg_col[nslot, nmi, pl.ds(g * 16, 16)] + qoff)
                return 0

            lax.fori_loop(0, CH // 16, addq, 0)

        def wait_prev_scatter():
            pltpu.make_async_copy(rowbuf.at[np_], acc.at[colq.at[np_]],
                                  sems_sg[np_]).wait()

        if first_pred is None:
            wait_prev_scatter()
        else:
            pl.when(first_pred)(wait_prev_scatter)

        if nxt is not None:
            pltpu.async_copy(zif.at[colq.at[np_]], rowbuf.at[np_],
                             sems_g[np_])
        pltpu.make_async_copy(zif.at[colq.at[gp]], rowbuf.at[gp],
                              sems_g[gp]).wait()

        def scale(k, _):
            i16s = jnp.full((16,), slot, jnp.int32)
            i16m = jnp.full((16,), mi, jnp.int32)
            i16k = jnp.full((16,), k, jnp.int32)
            vg = plsc.load_gather(stg_val, [i16s, i16m, i16k])
            for u in range(QH // 16):
                rowbuf[gp, k, pl.ds(u * 16, 16)] = (
                    rowbuf[gp, k, pl.ds(u * 16, 16)] * vg)
            return 0

        lax.fori_loop(0, CH, scale, 0)
        pltpu.async_copy(rowbuf.at[gp], acc.at[stg_row.at[slot, mi]],
                         sems_sg[gp], add=True)

    def zrow(i, _):
        for u in range(QH // 16):
            rowbuf[0, i, pl.ds(u * 16, 16)] = jnp.zeros((16,), jnp.float32)
        return 0

    lax.fori_loop(0, CH, zrow, 0)
    for i in range(RPT // CH):
        pltpu.sync_copy(rowbuf.at[0], acc.at[pl.ds(s * RPT + i * CH, CH)])
    plsc.subcore_barrier()

    pltpu.sync_copy(row3d.at[s * NSC], stg_row.at[0])
    pltpu.sync_copy(col3d.at[s * NSC], stg_col.at[0])
    pltpu.sync_copy(val3d.at[s * NSC], stg_val.at[0])

    def addq0(g, _):
        colq[0, pl.ds(g * 16, 16)] = stg_col[0, 0, pl.ds(g * 16, 16)] + qoff
        return 0

    lax.fori_loop(0, CH // 16, addq0, 0)
    pltpu.async_copy(zif.at[colq.at[0]], rowbuf.at[0], sems_g[0])

    def super_pair(t, _):
        emit_chunk(0, 0, 0, t > 0, (1, 0, False))
        issue_stage(2 * t + 1, 1)
        for mi in range(1, 4):
            emit_chunk(mi, 0, mi & 1, None, (mi + 1, 0, False))
        emit_chunk(4, 0, 0, None, (0, 1, True))
        emit_chunk(0, 1, 1, None, (1, 1, False))
        issue_stage(2 * t + 2, 0)
        for mi in range(1, 4):
            emit_chunk(mi, 1, (1 + mi) & 1, None, (mi + 1, 1, False))
        emit_chunk(4, 1, 1, None, (0, 0, True))
        return 0

    lax.fori_loop(0, (NSC - 1) // 2, super_pair, 0)

    for mi in range(4):
        emit_chunk(mi, 0, mi & 1, None, (mi + 1, 0, False))
    emit_chunk(4, 0, 0, None, None)
    pltpu.make_async_copy(rowbuf.at[0], acc.at[colq.at[0]],
                          sems_sg[0]).wait()
    plsc.subcore_barrier()

    for i in range(RPT // CH):
        r0 = s * RPT + i * CH
        pltpu.sync_copy(acc.at[pl.ds(r0, CH)], rowbuf.at[0])
        pltpu.sync_copy(rowbuf.at[0], zl2.at[c, pl.ds(r0, CH)])


def _run_sc2(zif, row3d, col3d, val3d):
    mesh = plsc.VectorSubcoreMesh(core_axis_name="c", subcore_axis_name="s")
    f = pl.kernel(
        _sc2_body,
        compiler_params=pltpu.CompilerParams(
            use_tc_tiling_on_sc=False, needs_layout_passes=False),
        out_type=[jax.ShapeDtypeStruct((2, NP, 2 * Q), jnp.float32)],
        mesh=mesh,
        scratch_types=[
            pltpu.VMEM((2, SB, CH), jnp.int32),
            pltpu.VMEM((2, SB, CH), jnp.int32),
            pltpu.VMEM((2, SB, CH), jnp.float32),
            pltpu.VMEM((2, CH), jnp.int32),
            pltpu.VMEM((2, CH, 2 * Q), jnp.float32),
            pltpu.VMEM_SHARED((NP, 2 * Q), jnp.float32),
            pltpu.SemaphoreType.DMA,
            pltpu.SemaphoreType.DMA,
            pltpu.SemaphoreType.DMA,
            pltpu.SemaphoreType.DMA,
            pltpu.SemaphoreType.DMA,
            pltpu.SemaphoreType.DMA,
        ],
    )
    return f(zif, row3d, col3d, val3d)


def _softplus(t):
    return jnp.maximum(t, 0.0) + jnp.log1p(jnp.exp(-jnp.abs(t)))


def _kb_body(hg_ref, hd_ref, mw1_ref, mb1_ref, mw2_ref, mb2_ref,
             sw1_ref, sb1_ref, sw2_ref, sb2_ref, ng_ref, nd_ref,
             fa_ref, fb_ref,
             zg_ref, zd_ref, mu_ref, std_ref,
             zi0_ref, zi1_ref):
    def enc(h):
        m1 = jnp.maximum(
            jnp.dot(h, mw1_ref[...], preferred_element_type=jnp.float32)
            + mb1_ref[...][None, :], 0.0)
        mean = (jnp.dot(m1, mw2_ref[...], preferred_element_type=jnp.float32)
                + mb2_ref[...][None, :])
        s1 = jnp.maximum(
            jnp.dot(h, sw1_ref[...], preferred_element_type=jnp.float32)
            + sb1_ref[...][None, :], 0.0)
        std = _softplus(
            jnp.dot(s1, sw2_ref[...], preferred_element_type=jnp.float32)
            + sb2_ref[...][None, :])
        return mean, std

    hg = jnp.concatenate(
        [jnp.maximum(hg_ref[q], 0.0) for q in range(4)], axis=1)
    hd = jnp.concatenate(
        [jnp.maximum(hd_ref[q], 0.0) for q in range(4)], axis=1)
    mu, std_g = enc(hg)
    mu_d, std_d = enc(hd)
    zg = ng_ref[...] * std_g + mu
    zd = nd_ref[...] * std_d + mu_d
    zi = fa_ref[...] * zg + fb_ref[...] * zd
    zg_ref[...] = zg
    zd_ref[...] = zd
    mu_ref[...] = mu
    std_ref[...] = std_g
    zi0_ref[...] = zi[:, 0:128]
    zi1_ref[...] = zi[:, 128:256]


def _run_kb(hg4, hd4, mw1, mb1, mw2, mb2, sw1, sb1, sw2, sb2, ng, nd, fa, fb):
    grid = N // RB
    h4spec = pl.BlockSpec((4, RB, Q), lambda i: (0, i, 0))
    wspec = pl.BlockSpec((256, 256), lambda i: (0, 0))
    bspec = pl.BlockSpec((256,), lambda i: (0,))
    nspec = pl.BlockSpec((RB, 256), lambda i: (i, 0))
    hspec = pl.BlockSpec((RB, 128), lambda i: (i, 0))
    return pl.pallas_call(
        _kb_body,
        grid=(grid,),
        in_specs=[h4spec, h4spec, wspec, bspec, wspec, bspec,
                  wspec, bspec, wspec, bspec, nspec, nspec, nspec, nspec],
        out_specs=[nspec, nspec, nspec, nspec, hspec, hspec],
        out_shape=[jax.ShapeDtypeStruct((N, 256), jnp.float32)] * 4
        + [jax.ShapeDtypeStruct((N, 128), jnp.float32)] * 2,
    )(hg4, hd4, mw1, mb1, mw2, mb2, sw1, sb1, sw2, sb2, ng, nd, fa, fb)


def _softmax(t):
    m = jnp.max(t, axis=1, keepdims=True)
    e = jnp.exp(t - m)
    return e / jnp.sum(e, axis=1, keepdims=True)


def _kd_body(zl2_ref, zg_ref, zd_ref, zi_ref, alpha_ref, hw_ref, hb_ref,
             zf_ref, qf_ref, qg_ref, qd_ref):
    zl = jnp.concatenate([zl2_ref[q] for q in range(2)], axis=1)
    alpha = alpha_ref[0]
    zi = zi_ref[...]
    zf = alpha * zl + (1.0 - alpha) * zi
    zf_ref[...] = zf
    hw = hw_ref[...]
    hb = hb_ref[...][None, :]
    qf_ref[...] = _softmax(
        jnp.dot(zf, hw, preferred_element_type=jnp.float32) + hb)
    qg_ref[...] = _softmax(
        jnp.dot(zg_ref[...], hw, preferred_element_type=jnp.float32) + hb)
    qd_ref[...] = _softmax(
        jnp.dot(zd_ref[...], hw, preferred_element_type=jnp.float32) + hb)


def _run_kd(zl2, zg, zd, zi, alpha, hw, hb):
    grid = N // RB
    nspec = pl.BlockSpec((RB, 256), lambda i: (i, 0))
    cspec = pl.BlockSpec((RB, C), lambda i: (i, 0))
    return pl.pallas_call(
        _kd_body,
        grid=(grid,),
        in_specs=[pl.BlockSpec((2, RB, 2 * Q), lambda i: (0, i, 0)),
                  nspec, nspec, nspec,
                  pl.BlockSpec(memory_space=pltpu.SMEM),
                  pl.BlockSpec((256, C), lambda i: (0, 0)),
                  pl.BlockSpec((C,), lambda i: (0,))],
        out_specs=[nspec, cspec, cspec, cspec],
        out_shape=[jax.ShapeDtypeStruct((N, 256), jnp.float32)]
        + [jax.ShapeDtypeStruct((N, C), jnp.float32)] * 3,
    )(zl2, zg, zd, zi, alpha, hw, hb)


def _ke_body(a_ref, b_ref, o_ref):
    o_ref[...] = lax.dot_general(
        a_ref[...], b_ref[...], (((1,), (1,)), ((), ())),
        preferred_element_type=jnp.float32)


def _run_ke(zg):
    cb = 1280
    return pl.pallas_call(
        _ke_body,
        grid=(N // RB, (N + cb - 1) // cb),
        in_specs=[pl.BlockSpec((RB, 256), lambda i, j: (i, 0)),
                  pl.BlockSpec((cb, 256), lambda i, j: (j, 0))],
        out_specs=pl.BlockSpec((RB, cb), lambda i, j: (i, j)),
        out_shape=jax.ShapeDtypeStruct((N, N), jnp.float32),
    )(zg, zg)


def kernel(x, adj_indices, adj_values, gcn_w, gcn_b, mean_w1, mean_b1,
           mean_w2, mean_b2, std_w1, std_b1, std_w2, std_b2,
           den_nb_w, den_nb_b, den_self_w, den_self_b, den_att_w, den_att_b,
           fus_a, fus_b, fus_alpha, head_w, head_b):
    row = adj_indices[0]
    col = adj_indices[1]
    row2d = row.reshape(NW, SB, CH)
    col2d = col.reshape(NW, SB, CH)
    val2d = adj_values.reshape(NW, SB, CH)

    rn = jax.random.uniform(jax.random.key(7), (E, 1), jnp.float32) + 1e-07
    lr2d = (jnp.log(rn) - jnp.log(1.0 - rn)).reshape(NW, SB, CH)
    ng = jax.random.normal(jax.random.key(11), (N, Z), jnp.float32)
    nd = jax.random.normal(jax.random.key(13), (N, Z), jnp.float32)

    wcat = jnp.concatenate([gcn_w, den_nb_w, den_self_w], axis=1)
    bcat = jnp.concatenate([gcn_b, den_nb_b, den_self_b], axis=0)
    a1 = den_att_w[:256].reshape(1, 256)
    a2 = den_att_w[256:].reshape(1, 256)

    xw4, s1r, s2r = _run_ka(x, wcat, bcat, a1, a2)
    xwf = xw4.reshape(4 * N, Q)
    s1 = s1r.reshape(N) + den_att_b[0]
    s2 = s2r.reshape(N)

    hg4, hd4, l0p, _nv = _run_sc1(xwf, row2d, col2d, val2d, lr2d, s1, s2)
    l0_loss = jnp.sum(l0p[:16]) / jnp.float32(E)

    zg, zd, mu, x_std, zi0, zi1 = _run_kb(
        hg4, hd4, mean_w1, mean_b1, mean_w2, mean_b2,
        std_w1, std_b1, std_w2, std_b2, ng, nd, fus_a, fus_b)

    zif = jnp.concatenate([zi0, zi1], axis=0)
    (zl2,) = _run_sc2(zif, row2d, col2d, val2d)

    zi = jnp.concatenate([zi0, zi1], axis=1)
    alpha = fus_alpha.reshape(1)

    adj_logits = _run_ke(zg)

    z_fused, q_fused, q_gen, q_den = _run_kd(zl2, zg, zd, zi, alpha,
                                             head_w, head_b)

    return (q_fused, q_gen, q_den, adj_logits, z_fused, mu, x_std, l0_loss)
